# baseline (speedup 1.0000x reference)
import functools

import jax
import jax.numpy as jnp
from jax.experimental import pallas as pl
from jax.experimental.pallas import tpu as pltpu

B, CIN, H, W = 16, 512, 40, 40
A, NC = 5, 80
CMID = 512
BN_EPS = 1e-5

WP = 48
ROWS = 40 * WP
PIX = H * W
XROWS = 42 * WP
MT = 240
PT = 200
NT = ROWS // MT
FPAD = 2080
NMISC = 32
CB = 96
NOUT = NMISC + A * CB


def _decode_kernel(x_ref, w1_ref, b1_ref, w2_ref, b2_ref, add_ref, scl_ref,
                   misc_ref, cls_ref, fp_ref, f3_ref):
    fp_ref[0:64, :] = jnp.zeros((64, CIN), jnp.bfloat16)
    fp_ref[64:64 + ROWS, :] = x_ref[0]
    fp_ref[64 + ROWS:, :] = jnp.zeros((FPAD - 64 - ROWS, CIN), jnp.bfloat16)
    f3_ref[:, 0:CIN] = fp_ref[pl.ds(15, XROWS), :]
    f3_ref[:, CIN:2 * CIN] = fp_ref[pl.ds(16, XROWS), :]
    f3_ref[:, 2 * CIN:3 * CIN] = fp_ref[pl.ds(17, XROWS), :]
    lane = jax.lax.broadcasted_iota(jnp.int32, (PT, NMISC), 1)
    use_exp = (lane < 20) & (lane % 4 >= 2)
    for m in range(NT):
        m0 = m * MT
        h = jnp.dot(f3_ref[pl.ds(m0, MT), :], w1_ref[0],
                    preferred_element_type=jnp.float32)
        h = h + jnp.dot(f3_ref[pl.ds(m0 + WP, MT), :], w1_ref[1],
                        preferred_element_type=jnp.float32)
        h = h + jnp.dot(f3_ref[pl.ds(m0 + 2 * WP, MT), :], w1_ref[2],
                        preferred_element_type=jnp.float32)
        h = h + b1_ref[0, :][None, :]
        h = h * (1.0 / (1.0 + jnp.exp(-h)))
        det = jnp.dot(h.astype(jnp.bfloat16), w2_ref[...],
                      preferred_element_type=jnp.float32)
        det = det + b2_ref[0, :][None, :]
        det = jnp.concatenate([det[k * WP: k * WP + W] for k in range(5)],
                              axis=0)
        p0 = m * PT
        misc = det[:, :NMISC]
        sig = 1.0 / (1.0 + jnp.exp(-misc))
        ex = jnp.exp(misc)
        v = jnp.where(use_exp, ex, sig)
        out = (v + add_ref[pl.ds(p0, PT), :]) * scl_ref[0, :][None, :]
        misc_ref[0, pl.ds(p0, PT), :] = out[:, :32]
        for a in range(A):
            blk = det[:, NMISC + a * CB: NMISC + (a + 1) * CB]
            e = jnp.exp(blk)
            sm = jnp.sum(e, axis=1, keepdims=True)
            cls_ref[0, pl.ds(p0, PT), pl.ds(a * NC, NC)] = (e * (1.0 / sm))[:, :NC]


@functools.partial(jax.jit, static_argnames=())
def kernel(input, conv1_w, bn_gamma, bn_beta, bn_mean, bn_var,
           conv2_w, conv2_b, anchors):
    f32, bf16 = jnp.float32, jnp.bfloat16

    scale = bn_gamma * jax.lax.rsqrt(bn_var + BN_EPS)
    b1 = (bn_beta - bn_mean * scale).reshape(1, CMID)
    w1 = (conv1_w * scale[:, None, None, None]).transpose(2, 3, 1, 0)
    w1 = w1.reshape(3, 3 * CIN, CMID).astype(bf16)

    w2 = conv2_w[:, :, 0, 0].T
    cols = []
    bias = []
    d = 5 + NC
    for a in range(A):
        for j in range(4):
            cols.append(a * d + j)
            if j == 2:
                bias.append(conv2_b[a * d + j] + jnp.log(anchors[a, 0]))
            elif j == 3:
                bias.append(conv2_b[a * d + j] + jnp.log(anchors[a, 1]))
            else:
                bias.append(conv2_b[a * d + j])
    for a in range(A):
        cols.append(a * d + 4)
        bias.append(conv2_b[a * d + 4])
    misc_idx = jnp.array(cols, jnp.int32)
    w2m = jnp.pad(w2[:, misc_idx], ((0, 0), (0, NMISC - 25)))
    b2m = jnp.pad(jnp.stack(bias), (0, NMISC - 25))
    cls_blocks = [jnp.pad(w2[:, a * d + 5: (a + 1) * d], ((0, 0), (0, CB - NC)))
                  for a in range(A)]
    b2c = [jnp.pad(conv2_b[a * d + 5: (a + 1) * d], (0, CB - NC),
                   constant_values=-1e9) for a in range(A)]
    w2p = jnp.concatenate([w2m] + cls_blocks, axis=1).astype(bf16)
    b2p = jnp.concatenate([b2m] + b2c).reshape(1, NOUT).astype(f32)

    p = jnp.arange(PIX, dtype=jnp.int32)
    gx = (p % W).astype(f32)
    gy = (p // W).astype(f32)
    lane = jnp.arange(NMISC)
    is_tx = (lane < 20) & (lane % 4 == 0)
    is_ty = (lane < 20) & (lane % 4 == 1)
    add = (jnp.where(is_tx[None, :], gx[:, None], 0.0)
           + jnp.where(is_ty[None, :], gy[:, None], 0.0))
    scl = jnp.where((lane < 20) & (lane % 4 < 2), 1.0 / W, 1.0
                    ).astype(f32).reshape(1, NMISC)

    xh = input.transpose(0, 2, 3, 1).astype(bf16)
    xp = jnp.pad(xh, ((0, 0), (0, 0), (0, WP - W), (0, 0)))
    fl = xp.reshape(B, ROWS, CIN)

    misc_o, cls_o = pl.pallas_call(
        _decode_kernel,
        grid=(B,),
        in_specs=[
            pl.BlockSpec((1, ROWS, CIN), lambda b: (b, 0, 0)),
            pl.BlockSpec((3, 3 * CIN, CMID), lambda b: (0, 0, 0)),
            pl.BlockSpec((1, CMID), lambda b: (0, 0)),
            pl.BlockSpec((CMID, NOUT), lambda b: (0, 0)),
            pl.BlockSpec((1, NOUT), lambda b: (0, 0)),
            pl.BlockSpec((PIX, NMISC), lambda b: (0, 0)),
            pl.BlockSpec((1, NMISC), lambda b: (0, 0)),
        ],
        out_specs=[
            pl.BlockSpec((1, PIX, 32), lambda b: (b, 0, 0)),
            pl.BlockSpec((1, PIX, A * NC), lambda b: (b, 0, 0)),
        ],
        out_shape=[jax.ShapeDtypeStruct((B, PIX, 32), f32),
                   jax.ShapeDtypeStruct((B, PIX, A * NC), f32)],
        compiler_params=pltpu.CompilerParams(
            dimension_semantics=("parallel",),
            vmem_limit_bytes=60 * 1024 * 1024,
        ),
        scratch_shapes=[pltpu.VMEM((FPAD, CIN), bf16),
                        pltpu.VMEM((XROWS, 3 * CIN), bf16)],
    )(fl, w1, b1, w2p, b2p, add, scl)

    boxes = misc_o[..., :20].reshape(B, H, W, A, 4)
    obj = misc_o[..., 20:25].reshape(B, H, W, A)
    classes = cls_o.reshape(B, H, W, A, NC)
    return boxes, obj, classes


# speedup vs baseline: 1.0362x; 1.0362x over previous
import functools

import jax
import jax.numpy as jnp
from jax.experimental import pallas as pl
from jax.experimental.pallas import tpu as pltpu

B, CIN, H, W = 16, 512, 40, 40
A, NC = 5, 80
CMID = 512
BN_EPS = 1e-5

WP = 48
ROWS = 40 * WP
PIX = H * W
XROWS = 42 * WP
MT = 240
PT = 200
NT = ROWS // MT
FPAD = 2080
NMISC = 128
CB = 128
NOUT = NMISC + A * CB


def _decode_kernel(x_ref, w1_ref, b1_ref, w2_ref, b2_ref, add_ref, scl_ref,
                   misc_ref, cls_ref, fp_ref, f3_ref):
    fp_ref[0:64, :] = jnp.zeros((64, CIN), jnp.bfloat16)
    fp_ref[64:64 + ROWS, :] = x_ref[0]
    fp_ref[64 + ROWS:, :] = jnp.zeros((FPAD - 64 - ROWS, CIN), jnp.bfloat16)
    f3_ref[:, 0:CIN] = fp_ref[pl.ds(15, XROWS), :]
    f3_ref[:, CIN:2 * CIN] = fp_ref[pl.ds(16, XROWS), :]
    f3_ref[:, 2 * CIN:3 * CIN] = fp_ref[pl.ds(17, XROWS), :]
    lane = jax.lax.broadcasted_iota(jnp.int32, (PT, NMISC), 1)
    use_exp = (lane < 20) & (lane % 4 >= 2)
    for m in range(NT):
        m0 = m * MT
        h = jnp.dot(f3_ref[pl.ds(m0, MT), :], w1_ref[0],
                    preferred_element_type=jnp.float32)
        h = h + jnp.dot(f3_ref[pl.ds(m0 + WP, MT), :], w1_ref[1],
                        preferred_element_type=jnp.float32)
        h = h + jnp.dot(f3_ref[pl.ds(m0 + 2 * WP, MT), :], w1_ref[2],
                        preferred_element_type=jnp.float32)
        h = h + b1_ref[0, :][None, :]
        h = h * (1.0 / (1.0 + jnp.exp(-h)))
        det = jnp.dot(h.astype(jnp.bfloat16), w2_ref[...],
                      preferred_element_type=jnp.float32)
        det = det + b2_ref[0, :][None, :]
        det = jnp.concatenate([det[k * WP: k * WP + W] for k in range(5)],
                              axis=0)
        p0 = m * PT
        misc = det[:, :NMISC]
        sig = 1.0 / (1.0 + jnp.exp(-misc))
        ex = jnp.exp(misc)
        v = jnp.where(use_exp, ex, sig)
        out = (v + add_ref[pl.ds(p0, PT), :]) * scl_ref[0, :][None, :]
        misc_ref[0, pl.ds(p0, PT), :] = out[:, :32]
        for a in range(A):
            blk = det[:, NMISC + a * CB: NMISC + (a + 1) * CB]
            e = jnp.exp(blk)
            sm = jnp.sum(e, axis=1, keepdims=True)
            cls_ref[0, pl.ds(p0, PT), pl.ds(a * NC, NC)] = (e * (1.0 / sm))[:, :NC]


@functools.partial(jax.jit, static_argnames=())
def kernel(input, conv1_w, bn_gamma, bn_beta, bn_mean, bn_var,
           conv2_w, conv2_b, anchors):
    f32, bf16 = jnp.float32, jnp.bfloat16

    scale = bn_gamma * jax.lax.rsqrt(bn_var + BN_EPS)
    b1 = (bn_beta - bn_mean * scale).reshape(1, CMID)
    w1 = (conv1_w * scale[:, None, None, None]).transpose(2, 3, 1, 0)
    w1 = w1.reshape(3, 3 * CIN, CMID).astype(bf16)

    w2 = conv2_w[:, :, 0, 0].T
    cols = []
    bias = []
    d = 5 + NC
    for a in range(A):
        for j in range(4):
            cols.append(a * d + j)
            if j == 2:
                bias.append(conv2_b[a * d + j] + jnp.log(anchors[a, 0]))
            elif j == 3:
                bias.append(conv2_b[a * d + j] + jnp.log(anchors[a, 1]))
            else:
                bias.append(conv2_b[a * d + j])
    for a in range(A):
        cols.append(a * d + 4)
        bias.append(conv2_b[a * d + 4])
    misc_idx = jnp.array(cols, jnp.int32)
    w2m = jnp.pad(w2[:, misc_idx], ((0, 0), (0, NMISC - 25)))
    b2m = jnp.pad(jnp.stack(bias), (0, NMISC - 25))
    cls_blocks = [jnp.pad(w2[:, a * d + 5: (a + 1) * d], ((0, 0), (0, CB - NC)))
                  for a in range(A)]
    b2c = [jnp.pad(conv2_b[a * d + 5: (a + 1) * d], (0, CB - NC),
                   constant_values=-1e9) for a in range(A)]
    w2p = jnp.concatenate([w2m] + cls_blocks, axis=1).astype(bf16)
    b2p = jnp.concatenate([b2m] + b2c).reshape(1, NOUT).astype(f32)

    p = jnp.arange(PIX, dtype=jnp.int32)
    gx = (p % W).astype(f32)
    gy = (p // W).astype(f32)
    lane = jnp.arange(NMISC)
    is_tx = (lane < 20) & (lane % 4 == 0)
    is_ty = (lane < 20) & (lane % 4 == 1)
    add = (jnp.where(is_tx[None, :], gx[:, None], 0.0)
           + jnp.where(is_ty[None, :], gy[:, None], 0.0))
    scl = jnp.where((lane < 20) & (lane % 4 < 2), 1.0 / W, 1.0
                    ).astype(f32).reshape(1, NMISC)

    xh = input.transpose(0, 2, 3, 1).astype(bf16)
    xp = jnp.pad(xh, ((0, 0), (0, 0), (0, WP - W), (0, 0)))
    fl = xp.reshape(B, ROWS, CIN)

    misc_o, cls_o = pl.pallas_call(
        _decode_kernel,
        grid=(B,),
        in_specs=[
            pl.BlockSpec((1, ROWS, CIN), lambda b: (b, 0, 0)),
            pl.BlockSpec((3, 3 * CIN, CMID), lambda b: (0, 0, 0)),
            pl.BlockSpec((1, CMID), lambda b: (0, 0)),
            pl.BlockSpec((CMID, NOUT), lambda b: (0, 0)),
            pl.BlockSpec((1, NOUT), lambda b: (0, 0)),
            pl.BlockSpec((PIX, NMISC), lambda b: (0, 0)),
            pl.BlockSpec((1, NMISC), lambda b: (0, 0)),
        ],
        out_specs=[
            pl.BlockSpec((1, PIX, 32), lambda b: (b, 0, 0)),
            pl.BlockSpec((1, PIX, A * NC), lambda b: (b, 0, 0)),
        ],
        out_shape=[jax.ShapeDtypeStruct((B, PIX, 32), f32),
                   jax.ShapeDtypeStruct((B, PIX, A * NC), f32)],
        compiler_params=pltpu.CompilerParams(
            dimension_semantics=("parallel",),
            vmem_limit_bytes=60 * 1024 * 1024,
        ),
        scratch_shapes=[pltpu.VMEM((FPAD, CIN), bf16),
                        pltpu.VMEM((XROWS, 3 * CIN), bf16)],
    )(fl, w1, b1, w2p, b2p, add, scl)

    boxes = misc_o[..., :20].reshape(B, H, W, A, 4)
    obj = misc_o[..., 20:25].reshape(B, H, W, A)
    classes = cls_o.reshape(B, H, W, A, NC)
    return boxes, obj, classes


# speedup vs baseline: 1.0374x; 1.0012x over previous
import functools

import jax
import jax.numpy as jnp
from jax.experimental import pallas as pl
from jax.experimental.pallas import tpu as pltpu

B, CIN, H, W = 16, 512, 40, 40
A, NC = 5, 80
CMID = 512
BN_EPS = 1e-5

WP = 48
ROWS = 40 * WP
PIX = H * W
XROWS = 42 * WP
MT = 240
PT = 200
NT = ROWS // MT
FPAD = 2080
NMISC = 128
CB = 128
NOUT = NMISC + A * CB


def _decode_kernel(x_ref, w1_ref, b1_ref, w2_ref, b2_ref, add_ref, scl_ref,
                   misc_ref, cls_ref, fp_ref, f3_ref):
    fp_ref[0:64, :] = jnp.zeros((64, CIN), jnp.bfloat16)
    fp_ref[64:64 + ROWS, :] = x_ref[0]
    fp_ref[64 + ROWS:, :] = jnp.zeros((FPAD - 64 - ROWS, CIN), jnp.bfloat16)
    f3_ref[:, 0:CIN] = fp_ref[pl.ds(15, XROWS), :]
    f3_ref[:, CIN:2 * CIN] = fp_ref[pl.ds(16, XROWS), :]
    f3_ref[:, 2 * CIN:3 * CIN] = fp_ref[pl.ds(17, XROWS), :]
    lane = jax.lax.broadcasted_iota(jnp.int32, (PT, NMISC), 1)
    use_exp = (lane < 20) & (lane % 4 >= 2)
    for m in range(NT):
        m0 = m * MT
        h = jnp.dot(f3_ref[pl.ds(m0, MT), :], w1_ref[0],
                    preferred_element_type=jnp.float32)
        h = h + jnp.dot(f3_ref[pl.ds(m0 + WP, MT), :], w1_ref[1],
                        preferred_element_type=jnp.float32)
        h = h + jnp.dot(f3_ref[pl.ds(m0 + 2 * WP, MT), :], w1_ref[2],
                        preferred_element_type=jnp.float32)
        h = h + b1_ref[0, :][None, :]
        h = h * (1.0 / (1.0 + jnp.exp(-h)))
        det = jnp.dot(h.astype(jnp.bfloat16), w2_ref[...],
                      preferred_element_type=jnp.float32)
        det = det + b2_ref[0, :][None, :]
        det = jnp.concatenate([det[k * WP: k * WP + W] for k in range(5)],
                              axis=0)
        p0 = m * PT
        misc = det[:, :NMISC]
        sig = 1.0 / (1.0 + jnp.exp(-misc))
        ex = jnp.exp(misc)
        v = jnp.where(use_exp, ex, sig)
        out = (v + add_ref[pl.ds(p0, PT), :]) * scl_ref[0, :][None, :]
        misc_ref[0, pl.ds(p0, PT), :] = out[:, :32]
        for a in range(A):
            blk = det[:, NMISC + a * CB: NMISC + (a + 1) * CB]
            e = jnp.exp(blk)
            sm = jnp.sum(e, axis=1, keepdims=True)
            cls_ref[0, pl.ds(p0, PT), pl.ds(a * NC, NC)] = (e * (1.0 / sm))[:, :NC]


@functools.partial(jax.jit, static_argnames=())
def kernel(input, conv1_w, bn_gamma, bn_beta, bn_mean, bn_var,
           conv2_w, conv2_b, anchors):
    f32, bf16 = jnp.float32, jnp.bfloat16

    scale = bn_gamma * jax.lax.rsqrt(bn_var + BN_EPS)
    b1 = (bn_beta - bn_mean * scale).reshape(1, CMID)
    w1 = (conv1_w * scale[:, None, None, None]).transpose(2, 3, 1, 0)
    w1 = w1.reshape(3, 3 * CIN, CMID).astype(bf16)

    w2 = conv2_w[:, :, 0, 0].T
    cols = []
    bias = []
    d = 5 + NC
    for a in range(A):
        for j in range(4):
            cols.append(a * d + j)
            if j == 2:
                bias.append(conv2_b[a * d + j] + jnp.log(anchors[a, 0]))
            elif j == 3:
                bias.append(conv2_b[a * d + j] + jnp.log(anchors[a, 1]))
            else:
                bias.append(conv2_b[a * d + j])
    for a in range(A):
        cols.append(a * d + 4)
        bias.append(conv2_b[a * d + 4])
    misc_idx = jnp.array(cols, jnp.int32)
    w2m = jnp.pad(w2[:, misc_idx], ((0, 0), (0, NMISC - 25)))
    b2m = jnp.pad(jnp.stack(bias), (0, NMISC - 25))
    cls_blocks = [jnp.pad(w2[:, a * d + 5: (a + 1) * d], ((0, 0), (0, CB - NC)))
                  for a in range(A)]
    b2c = [jnp.pad(conv2_b[a * d + 5: (a + 1) * d], (0, CB - NC),
                   constant_values=-1e9) for a in range(A)]
    w2p = jnp.concatenate([w2m] + cls_blocks, axis=1).astype(bf16)
    b2p = jnp.concatenate([b2m] + b2c).reshape(1, NOUT).astype(f32)

    p = jnp.arange(PIX, dtype=jnp.int32)
    gx = (p % W).astype(f32)
    gy = (p // W).astype(f32)
    lane = jnp.arange(NMISC)
    is_tx = (lane < 20) & (lane % 4 == 0)
    is_ty = (lane < 20) & (lane % 4 == 1)
    add = (jnp.where(is_tx[None, :], gx[:, None], 0.0)
           + jnp.where(is_ty[None, :], gy[:, None], 0.0))
    scl = jnp.where((lane < 20) & (lane % 4 < 2), 1.0 / W, 1.0
                    ).astype(f32).reshape(1, NMISC)

    xh = input.transpose(0, 2, 3, 1).astype(bf16)
    xp = jnp.pad(xh, ((0, 0), (0, 0), (0, WP - W), (0, 0)))
    fl = xp.reshape(B, ROWS, CIN)

    misc_o, cls_o = pl.pallas_call(
        _decode_kernel,
        grid=(B,),
        in_specs=[
            pl.BlockSpec((1, ROWS, CIN), lambda b: (b, 0, 0)),
            pl.BlockSpec((3, 3 * CIN, CMID), lambda b: (0, 0, 0)),
            pl.BlockSpec((1, CMID), lambda b: (0, 0)),
            pl.BlockSpec((CMID, NOUT), lambda b: (0, 0)),
            pl.BlockSpec((1, NOUT), lambda b: (0, 0)),
            pl.BlockSpec((PIX, NMISC), lambda b: (0, 0)),
            pl.BlockSpec((1, NMISC), lambda b: (0, 0)),
        ],
        out_specs=[
            pl.BlockSpec((1, PIX, 32), lambda b: (b, 0, 0)),
            pl.BlockSpec((1, PIX, A * NC), lambda b: (b, 0, 0)),
        ],
        out_shape=[jax.ShapeDtypeStruct((B, PIX, 32), f32),
                   jax.ShapeDtypeStruct((B, PIX, A * NC), f32)],
        compiler_params=pltpu.CompilerParams(
            dimension_semantics=("parallel",),
            allow_input_fusion=(True, False, False, False, False, False, False),
            vmem_limit_bytes=60 * 1024 * 1024,
        ),
        scratch_shapes=[pltpu.VMEM((FPAD, CIN), bf16),
                        pltpu.VMEM((XROWS, 3 * CIN), bf16)],
    )(fl, w1, b1, w2p, b2p, add, scl)

    boxes = misc_o[..., :20].reshape(B, H, W, A, 4)
    obj = misc_o[..., 20:25].reshape(B, H, W, A)
    classes = cls_o.reshape(B, H, W, A, NC)
    return boxes, obj, classes
